# baseline (speedup 1.0000x reference)
import functools

import jax
import jax.numpy as jnp
from jax import lax
from jax.experimental import pallas as pl
from jax.experimental.pallas import tpu as pltpu
from jax.experimental.pallas import tpu_sc as plsc

NC = 2
NS = 16
CHUNK = 64
SUB = 24
NPHASE = 2
LANES = 16


def _make_sc_scatter(nch, n_pad, d, dtype):
    rows_per_tile = n_pad // NS
    stage_rows = CHUNK
    d2 = d // NPHASE
    mesh = plsc.VectorSubcoreMesh(core_axis_name="c", subcore_axis_name="s")

    @functools.partial(
        pl.kernel,
        out_type=jax.ShapeDtypeStruct((NC, n_pad, d), dtype),
        mesh=mesh,
        scratch_types=[
            pltpu.VMEM((SUB, CHUNK), jnp.int32),
            pltpu.VMEM((SUB, CHUNK), jnp.int32),
            pltpu.VMEM((SUB, CHUNK), dtype),
            pltpu.VMEM((CHUNK, d2), dtype),
            pltpu.VMEM((CHUNK, d2), dtype),
            pltpu.VMEM((CHUNK, d2), dtype),
            pltpu.VMEM_SHARED((n_pad, d2), dtype),
            pltpu.VMEM_SHARED((n_pad, d2), dtype),
            pltpu.SemaphoreType.DMA,
            pltpu.SemaphoreType.DMA,
            pltpu.SemaphoreType.DMA,
            pltpu.SemaphoreType.DMA,
            pltpu.SemaphoreType.DMA,
            pltpu.SemaphoreType.DMA,
        ],
        compiler_params=pltpu.CompilerParams(use_tc_tiling_on_sc=False),
    )
    def sc_scatter(gidx_hbm, sidx_hbm, attr_hbm, x_hbm, out_hbm,
                   gidx_v, sidx_v, attr_v, rows_0, rows_1, rows_2, xs_sh,
                   acc_sh, gsem_0, gsem_1, gsem_2, ssem_0, ssem_1, ssem_2):
        c = lax.axis_index("c")
        s = lax.axis_index("s")
        base = s * rows_per_tile
        bufs = [rows_0, rows_1, rows_2]
        gsems = [gsem_0, gsem_1, gsem_2]
        ssems = [ssem_0, ssem_1, ssem_2]

        def scale(rows_v, j):
            def edge_body(g, icarry):
                a16 = attr_v[j, pl.ds(g * LANES, LANES)]
                for t in range(LANES):
                    av = jnp.full((LANES,), a16[t], dtype)
                    i = g * LANES + t
                    for q in range(d2 // LANES):
                        sl = pl.ds(q * LANES, LANES)
                        rows_v[i, sl] = rows_v[i, sl] * av
                return icarry
            lax.fori_loop(0, CHUNK // LANES, edge_body, 0)

        for p in range(NPHASE):
            pltpu.sync_copy(x_hbm.at[pl.ds(base, rows_per_tile),
                                     pl.ds(p * d2, d2)],
                            xs_sh.at[pl.ds(base, rows_per_tile)])

            def zero_row(i, carry):
                for q in range(d2 // LANES):
                    rows_0[i, pl.ds(q * LANES, LANES)] = jnp.zeros((LANES,), dtype)
                return carry
            lax.fori_loop(0, stage_rows, zero_row, 0)
            for k in range(rows_per_tile // stage_rows):
                pltpu.sync_copy(rows_0,
                                acc_sh.at[pl.ds(base + k * stage_rows, stage_rows)])
            plsc.subcore_barrier()

            def swait(P):
                pltpu.make_async_copy(bufs[P], acc_sh.at[sidx_v.at[0]],
                                      ssems[P]).wait()

            def step(j, P, first, prefetch_j):
                pltpu.make_async_copy(xs_sh.at[gidx_v.at[j]], bufs[P],
                                      gsems[P]).wait()
                scale(bufs[P], j)
                pltpu.async_copy(bufs[P], acc_sh.at[sidx_v.at[j]], ssems[P],
                                 add=True)
                if prefetch_j is not None:
                    Q = (P + 2) % 3
                    if not first:
                        swait(Q)
                    pltpu.async_copy(xs_sh.at[gidx_v.at[prefetch_j]], bufs[Q],
                                     gsems[Q])

            def block_body(b, bcarry):
                pltpu.sync_copy(gidx_hbm.at[c, s, pl.ds(b * SUB, SUB)], gidx_v)
                pltpu.sync_copy(sidx_hbm.at[c, s, pl.ds(b * SUB, SUB)], sidx_v)
                pltpu.sync_copy(attr_hbm.at[s, pl.ds(b * SUB, SUB)], attr_v)
                pltpu.async_copy(xs_sh.at[gidx_v.at[0]], rows_0, gsem_0)
                pltpu.async_copy(xs_sh.at[gidx_v.at[1]], rows_1, gsem_1)

                step(0, 0, True, 2)
                step(1, 1, False, 3)

                def triple(j2, carry):
                    j = j2 * 3 + 2
                    step(j, 2, False, j + 2)
                    step(j + 1, 0, False, j + 3)
                    step(j + 2, 1, False, j + 4)
                    return carry
                lax.fori_loop(0, (SUB - 6) // 3, triple, 0)
                step(SUB - 4, (SUB - 4) % 3, False, SUB - 2)
                step(SUB - 3, (SUB - 3) % 3, False, SUB - 1)
                step(SUB - 2, (SUB - 2) % 3, False, None)
                step(SUB - 1, (SUB - 1) % 3, False, None)
                swait((SUB - 3) % 3)
                swait((SUB - 2) % 3)
                swait((SUB - 1) % 3)
                return bcarry
            lax.fori_loop(0, nch // SUB, block_body, 0)
            plsc.subcore_barrier()

            for k in range(rows_per_tile // stage_rows):
                off = base + k * stage_rows
                pltpu.sync_copy(acc_sh.at[pl.ds(off, stage_rows)], rows_0)
                pltpu.sync_copy(rows_0, out_hbm.at[c, pl.ds(off, stage_rows),
                                                   pl.ds(p * d2, d2)])

    return sc_scatter


def _mlp_body(mi_ref, mo_ref, x_ref, w1a_ref, w1b_ref, w1c_ref, b1_ref,
              w2_ref, b2_ref, o_ref):
    acc = jnp.dot(mi_ref[...], w1a_ref[...], preferred_element_type=jnp.float32)
    acc = acc + jnp.dot(mo_ref[...], w1b_ref[...], preferred_element_type=jnp.float32)
    acc = acc + jnp.dot(x_ref[...], w1c_ref[...], preferred_element_type=jnp.float32)
    h = jnp.tanh(acc + b1_ref[...])
    o = jnp.dot(h, w2_ref[...], preferred_element_type=jnp.float32) + b2_ref[...]
    o_ref[...] = jnp.tanh(o)


def _mlp(mi, mo, x, W1, b1, W2, b2):
    n, d = x.shape
    blk = 400
    grid = n // blk
    row_spec = pl.BlockSpec((blk, d), lambda i: (i, 0))
    full = lambda shape: pl.BlockSpec(shape, lambda i: tuple(0 for _ in shape))
    return pl.pallas_call(
        _mlp_body,
        grid=(grid,),
        in_specs=[
            row_spec, row_spec, row_spec,
            full((d, d)), full((d, d)), full((d, d)), full((1, d)),
            full((d, d)), full((1, d)),
        ],
        out_specs=row_spec,
        out_shape=jax.ShapeDtypeStruct((n, d), x.dtype),
    )(mi, mo, x, W1[:d], W1[d:2 * d], W1[2 * d:], b1.reshape(1, d),
      W2, b2.reshape(1, d))


def kernel(x, edge_index, edge_attr, W1, b1, W2, b2):
    n, d = x.shape
    e = edge_index.shape[1]
    per_tile = -(-e // NS)
    nch = -(-per_tile // (CHUNK * SUB)) * SUB
    e_pad = NS * nch * CHUNK
    pad = e_pad - e

    row = edge_index[0]
    col = edge_index[1]
    attr = edge_attr[:, 0]
    if pad:
        zi = jnp.zeros((pad,), jnp.int32)
        row = jnp.concatenate([row, zi])
        col = jnp.concatenate([col, zi])
        attr = jnp.concatenate([attr, jnp.zeros((pad,), attr.dtype)])

    gidx = jnp.stack([row, col]).reshape(NC, NS, nch, CHUNK)
    sidx = jnp.stack([col, row]).reshape(NC, NS, nch, CHUNK)
    attr_r = attr.reshape(NS, nch, CHUNK)

    rpt = -(-(-(-n // NS)) // CHUNK) * CHUNK
    n_pad = NS * rpt
    x_pad = jnp.zeros((n_pad, d), x.dtype).at[:n].set(x)
    mimo = _make_sc_scatter(nch, n_pad, d, x.dtype)(gidx, sidx, attr_r, x_pad)
    return _mlp(mimo[0, :n], mimo[1, :n], x, W1, b1, W2, b2)


# speedup vs baseline: 1.3434x; 1.3434x over previous
import functools

import jax
import jax.numpy as jnp
from jax import lax
from jax.experimental import pallas as pl
from jax.experimental.pallas import tpu as pltpu
from jax.experimental.pallas import tpu_sc as plsc

NC = 2
NS = 16
CHUNK = 128
SUB = 24
NPHASE = 2
LANES = 16


def _make_sc_scatter(nch, n_pad, d, dtype):
    rows_per_tile = n_pad // NS
    stage_rows = CHUNK
    d2 = d // NPHASE
    mesh = plsc.VectorSubcoreMesh(core_axis_name="c", subcore_axis_name="s")

    @functools.partial(
        pl.kernel,
        out_type=jax.ShapeDtypeStruct((NC, n_pad, d), dtype),
        mesh=mesh,
        scratch_types=[
            pltpu.VMEM((SUB, CHUNK), jnp.int32),
            pltpu.VMEM((SUB, CHUNK), jnp.int32),
            pltpu.VMEM((SUB, CHUNK), dtype),
            pltpu.VMEM((CHUNK, d2), jnp.bfloat16),
            pltpu.VMEM((CHUNK, d2), jnp.bfloat16),
            pltpu.VMEM((CHUNK, d2), jnp.bfloat16),
            pltpu.VMEM((CHUNK, d2), dtype),
            pltpu.VMEM((CHUNK, d2), dtype),
            pltpu.VMEM((CHUNK, d2), dtype),
            pltpu.VMEM_SHARED((n_pad, d2), jnp.bfloat16),
            pltpu.VMEM_SHARED((n_pad, d2), dtype),
            pltpu.SemaphoreType.DMA,
            pltpu.SemaphoreType.DMA,
            pltpu.SemaphoreType.DMA,
            pltpu.SemaphoreType.DMA,
            pltpu.SemaphoreType.DMA,
            pltpu.SemaphoreType.DMA,
        ],
        compiler_params=pltpu.CompilerParams(use_tc_tiling_on_sc=False,
                                             needs_layout_passes=False),
    )
    def sc_scatter(gidx_hbm, sidx_hbm, attr_hbm, x_hbm, out_hbm,
                   gidx_v, sidx_v, attr_v, gb_0, gb_1, gb_2, sb_0, sb_1, sb_2,
                   xs_sh, acc_sh, gsem_0, gsem_1, gsem_2, ssem_0, ssem_1,
                   ssem_2):
        c = lax.axis_index("c")
        s = lax.axis_index("s")
        base = s * rows_per_tile
        gbufs = [gb_0, gb_1, gb_2]
        sbufs = [sb_0, sb_1, sb_2]
        gsems = [gsem_0, gsem_1, gsem_2]
        ssems = [ssem_0, ssem_1, ssem_2]

        def scale(P, j):
            gb, sb = gbufs[P], sbufs[P]
            def edge_body(g, icarry):
                a16 = attr_v[j, pl.ds(g * LANES, LANES)]
                for t in range(LANES):
                    av = jnp.full((LANES,), a16[t], dtype)
                    i = g * LANES + t
                    for g2 in range(d2 // (2 * LANES)):
                        v32 = gb[i, pl.ds(g2 * 2 * LANES, 2 * LANES)]
                        lo, hi = plsc.unpack(
                            v32, format=plsc.PackFormat.INTERLEAVED)
                        sb[i, pl.ds(g2 * 2 * LANES, LANES)] = lo * av
                        sb[i, pl.ds(g2 * 2 * LANES + LANES, LANES)] = hi * av
                return icarry
            lax.fori_loop(0, CHUNK // LANES, edge_body, 0)

        for p in range(NPHASE):
            pltpu.sync_copy(x_hbm.at[pl.ds(base, rows_per_tile),
                                     pl.ds(p * d2, d2)],
                            xs_sh.at[pl.ds(base, rows_per_tile)])

            def zero_row(i, carry):
                for q in range(d2 // LANES):
                    sb_0[i, pl.ds(q * LANES, LANES)] = jnp.zeros((LANES,), dtype)
                return carry
            lax.fori_loop(0, stage_rows, zero_row, 0)
            for k in range(rows_per_tile // stage_rows):
                pltpu.sync_copy(sb_0,
                                acc_sh.at[pl.ds(base + k * stage_rows, stage_rows)])
            plsc.subcore_barrier()

            def swait(P):
                pltpu.make_async_copy(sbufs[P], acc_sh.at[sidx_v.at[0]],
                                      ssems[P]).wait()

            def step(j, P, first, prefetch_j):
                pltpu.make_async_copy(xs_sh.at[gidx_v.at[j]], gbufs[P],
                                      gsems[P]).wait()
                if prefetch_j is not None:
                    Q = (P + 2) % 3
                    pltpu.async_copy(xs_sh.at[gidx_v.at[prefetch_j]], gbufs[Q],
                                     gsems[Q])
                if not first:
                    swait(P)
                scale(P, j)
                pltpu.async_copy(sbufs[P], acc_sh.at[sidx_v.at[j]], ssems[P],
                                 add=True)

            def block_body(b, bcarry):
                pltpu.sync_copy(gidx_hbm.at[c, s, pl.ds(b * SUB, SUB)], gidx_v)
                pltpu.sync_copy(sidx_hbm.at[c, s, pl.ds(b * SUB, SUB)], sidx_v)
                pltpu.sync_copy(attr_hbm.at[s, pl.ds(b * SUB, SUB)], attr_v)
                pltpu.async_copy(xs_sh.at[gidx_v.at[0]], gb_0, gsem_0)
                pltpu.async_copy(xs_sh.at[gidx_v.at[1]], gb_1, gsem_1)

                step(0, 0, True, 2)
                step(1, 1, True, 3)
                step(2, 2, True, 4)

                def triple(j2, carry):
                    j = j2 * 3 + 3
                    step(j, 0, False, j + 2)
                    step(j + 1, 1, False, j + 3)
                    step(j + 2, 2, False, j + 4)
                    return carry
                lax.fori_loop(0, (SUB - 6) // 3, triple, 0)
                step(SUB - 3, (SUB - 3) % 3, False, SUB - 1)
                step(SUB - 2, (SUB - 2) % 3, False, None)
                step(SUB - 1, (SUB - 1) % 3, False, None)
                swait((SUB - 3) % 3)
                swait((SUB - 2) % 3)
                swait((SUB - 1) % 3)
                return bcarry
            lax.fori_loop(0, nch // SUB, block_body, 0)
            plsc.subcore_barrier()

            for k in range(rows_per_tile // stage_rows):
                off = base + k * stage_rows
                pltpu.sync_copy(acc_sh.at[pl.ds(off, stage_rows)], sb_0)
                pltpu.sync_copy(sb_0, out_hbm.at[c, pl.ds(off, stage_rows),
                                                 pl.ds(p * d2, d2)])

    return sc_scatter


def _mlp_body(mi_ref, mo_ref, x_ref, w1a_ref, w1b_ref, w1c_ref, b1_ref,
              w2_ref, b2_ref, o_ref):
    acc = jnp.dot(mi_ref[...], w1a_ref[...], preferred_element_type=jnp.float32)
    acc = acc + jnp.dot(mo_ref[...], w1b_ref[...], preferred_element_type=jnp.float32)
    acc = acc + jnp.dot(x_ref[...], w1c_ref[...], preferred_element_type=jnp.float32)
    h = jnp.tanh(acc + b1_ref[...])
    o = jnp.dot(h, w2_ref[...], preferred_element_type=jnp.float32) + b2_ref[...]
    o_ref[...] = jnp.tanh(o)


def _mlp(mi, mo, x, W1, b1, W2, b2):
    n, d = x.shape
    blk = 400
    grid = n // blk
    row_spec = pl.BlockSpec((blk, d), lambda i: (i, 0))
    full = lambda shape: pl.BlockSpec(shape, lambda i: tuple(0 for _ in shape))
    return pl.pallas_call(
        _mlp_body,
        grid=(grid,),
        in_specs=[
            row_spec, row_spec, row_spec,
            full((d, d)), full((d, d)), full((d, d)), full((1, d)),
            full((d, d)), full((1, d)),
        ],
        out_specs=row_spec,
        out_shape=jax.ShapeDtypeStruct((n, d), x.dtype),
    )(mi, mo, x, W1[:d], W1[d:2 * d], W1[2 * d:], b1.reshape(1, d),
      W2, b2.reshape(1, d))


def kernel(x, edge_index, edge_attr, W1, b1, W2, b2):
    n, d = x.shape
    e = edge_index.shape[1]
    per_tile = -(-e // NS)
    nch = -(-per_tile // (CHUNK * SUB)) * SUB
    e_pad = NS * nch * CHUNK
    pad = e_pad - e

    row = edge_index[0]
    col = edge_index[1]
    attr = edge_attr[:, 0]
    if pad:
        zi = jnp.zeros((pad,), jnp.int32)
        row = jnp.concatenate([row, zi])
        col = jnp.concatenate([col, zi])
        attr = jnp.concatenate([attr, jnp.zeros((pad,), attr.dtype)])

    gidx = jnp.stack([row, col]).reshape(NC, NS, nch, CHUNK)
    sidx = jnp.stack([col, row]).reshape(NC, NS, nch, CHUNK)
    attr_r = attr.reshape(NS, nch, CHUNK)

    rpt = -(-(-(-n // NS)) // CHUNK) * CHUNK
    n_pad = NS * rpt
    x_pad = jnp.zeros((n_pad, d), x.dtype).at[:n].set(x)
    x_swz = jnp.transpose(x_pad.reshape(n_pad, d // 32, 2, LANES),
                          (0, 1, 3, 2)).reshape(n_pad, d)
    x_bf = x_swz.astype(jnp.bfloat16)
    mimo = _make_sc_scatter(nch, n_pad, d, x.dtype)(gidx, sidx, attr_r, x_bf)
    return _mlp(mimo[0, :n], mimo[1, :n], x, W1, b1, W2, b2)
